# baseline (speedup 1.0000x reference)
import numpy as np
import jax
import jax.numpy as jnp
from jax import lax
from jax.experimental import pallas as pl
from jax.experimental.pallas import tpu as pltpu

_NB = 8192
_C = 101
_NPIX = 4 * 512 * 512

_EDGES = np.arange(0, 0.5, 0.005).astype(np.float32)
_NOISY = np.arange(-0.0101, 0.0101, 0.0002).astype(np.float32)
_LOWER = np.concatenate([[-np.inf], _EDGES]).astype(np.float32)
_UPPER = np.concatenate([_EDGES, [np.inf]]).astype(np.float32)

_U32 = lambda v: np.uint32(v)
_TINY = np.float32(np.finfo(np.float32).tiny)
_KS = (np.uint32(0), np.uint32(42), np.uint32(np.uint32(42) ^ np.uint32(0x1BD11BDA)))


def _kern(x_ref, lognm_ref, lower_ref, upper_ref, noisy_ref, ctab_ref, out_ref):
    i = pl.program_id(0)
    xv = x_ref[0]

    onehot = ((xv >= lower_ref[...]) & (xv < upper_ref[...])).astype(jnp.float32)

    logits = lax.dot_general(
        lognm_ref[...], onehot,
        dimension_numbers=(((0,), (0,)), ((), ())),
        precision=lax.Precision.HIGHEST,
        preferred_element_type=jnp.float32,
    )

    off = (i * (_NB * _C)).astype(jnp.uint32)
    x1 = ctab_ref[...] + off
    x0 = x1
    x1 = ((x1 << _U32(13)) | (x1 >> _U32(19))) ^ x0
    for rot in (15, 26, 6):
        x0 = x0 + x1
        x1 = ((x1 << _U32(rot)) | (x1 >> _U32(32 - rot))) ^ x0
    x0 = x0 + _KS[1]
    x1 = x1 + _U32(_KS[2] + _U32(1))
    rotations = ((13, 15, 26, 6), (17, 29, 16, 24))
    for r in range(1, 5):
        for rot in rotations[r % 2]:
            x0 = x0 + x1
            x1 = ((x1 << _U32(rot)) | (x1 >> _U32(32 - rot))) ^ x0
        x0 = x0 + _KS[(r + 1) % 3]
        x1 = x1 + _U32(_KS[(r + 2) % 3] + _U32(r + 1))
    bits = x0 ^ x1

    fb = lax.bitcast_convert_type(
        (bits >> _U32(9)) | _U32(0x3F800000), jnp.float32) - jnp.float32(1.0)
    u = jnp.maximum(fb, _TINY)
    g = -jnp.log(-jnp.log(u))

    v = g + logits
    mx = jnp.max(v, axis=0, keepdims=True)
    picked = jnp.where(v == mx, noisy_ref[...], jnp.float32(np.inf))
    out_ref[0] = jnp.min(picked, axis=0, keepdims=True)


def kernel(x, noise_matrix):
    nblk = _NPIX // _NB
    lognm = jnp.log(noise_matrix)
    lower = jnp.asarray(np.broadcast_to(_LOWER[:, None], (_C, _NB)))
    upper = jnp.asarray(np.broadcast_to(_UPPER[:, None], (_C, _NB)))
    noisy = jnp.asarray(np.broadcast_to(_NOISY[:, None], (_C, _NB)))
    ctab = jnp.asarray(
        101 * np.arange(_NB, dtype=np.uint32)[None, :]
        + np.arange(_C, dtype=np.uint32)[:, None]
        + np.uint32(42)
    )

    ndev = jax.device_count()
    if ndev > 1 and x.shape[0] % ndev == 0 and jax.process_count() == 1:
        mesh = jax.sharding.Mesh(np.array(jax.devices()), ("d",))
        nloc = nblk // ndev

        def _shard_fn(x_s, lognm_s, lower_s, upper_s, noisy_s, ctab_s):
            base = lax.axis_index("d").astype(jnp.uint32) * np.uint32(
                nloc * _NB * _C
            )
            out_s = _call_pallas(
                x_s.reshape(nloc, 1, _NB), lognm_s, lower_s, upper_s,
                noisy_s, ctab_s + base, nloc,
            )
            return out_s.reshape(x.shape[0] // ndev, *x.shape[1:])

        P = jax.sharding.PartitionSpec
        return jax.shard_map(
            _shard_fn,
            mesh=mesh,
            in_specs=(
                P("d", None, None), P(None, None), P(None, None),
                P(None, None), P(None, None), P(None, None),
            ),
            out_specs=P("d", None, None),
            check_vma=False,
        )(x, lognm, lower, upper, noisy, ctab)
    xb = x.reshape(nblk, 1, _NB)
    out = _call_pallas(xb, lognm, lower, upper, noisy, ctab, nblk)
    return out.reshape(x.shape)


def _call_pallas(xb, lognm, lower, upper, noisy, ctab, nblk):
    return pl.pallas_call(
        _kern,
        grid=(nblk,),
        in_specs=[
            pl.BlockSpec((1, 1, _NB), lambda i: (i, 0, 0)),
            pl.BlockSpec((_C, _C), lambda i: (0, 0)),
            pl.BlockSpec((_C, _NB), lambda i: (0, 0)),
            pl.BlockSpec((_C, _NB), lambda i: (0, 0)),
            pl.BlockSpec((_C, _NB), lambda i: (0, 0)),
            pl.BlockSpec((_C, _NB), lambda i: (0, 0)),
        ],
        out_specs=pl.BlockSpec((1, 1, _NB), lambda i: (i, 0, 0)),
        out_shape=jax.ShapeDtypeStruct((nblk, 1, _NB), jnp.float32),
        compiler_params=pltpu.CompilerParams(
            dimension_semantics=("arbitrary",),
        ),
    )(xb, lognm, lower, upper, noisy, ctab)


# speedup vs baseline: 1.3635x; 1.3635x over previous
import numpy as np
import jax
import jax.numpy as jnp
from jax import lax
from jax.experimental import pallas as pl
from jax.experimental.pallas import tpu as pltpu

_NB = 8192
_C = 101
_NPIX = 4 * 512 * 512

_EDGES = np.arange(0, 0.5, 0.005).astype(np.float32)
_NOISY = np.arange(-0.0101, 0.0101, 0.0002).astype(np.float32)
_LOWER = np.concatenate([[-np.inf], _EDGES]).astype(np.float32)
_UPPER = np.concatenate([_EDGES, [np.inf]]).astype(np.float32)

_U32 = lambda v: np.uint32(v)
_TINY = np.float32(np.finfo(np.float32).tiny)
_KS = (np.uint32(0), np.uint32(42), np.uint32(np.uint32(42) ^ np.uint32(0x1BD11BDA)))


def _kern(x_ref, lognm_ref, lower_ref, upper_ref, noisy_ref, ctab_ref, out_ref):
    i = pl.program_id(0)
    xv = x_ref[0]

    onehot = ((xv >= lower_ref[...]) & (xv < upper_ref[...])).astype(jnp.float32)

    logits = lax.dot_general(
        lognm_ref[...], onehot,
        dimension_numbers=(((0,), (0,)), ((), ())),
        precision=lax.Precision.HIGHEST,
        preferred_element_type=jnp.float32,
    )

    off = (i * (_NB * _C)).astype(jnp.uint32)
    x1 = ctab_ref[...] + off
    x0 = x1
    x1 = ((x1 << _U32(13)) | (x1 >> _U32(19))) ^ x0
    for rot in (15, 26, 6):
        x0 = x0 + x1
        x1 = ((x1 << _U32(rot)) | (x1 >> _U32(32 - rot))) ^ x0
    x0 = x0 + _KS[1]
    x1 = x1 + _U32(_KS[2] + _U32(1))
    rotations = ((13, 15, 26, 6), (17, 29, 16, 24))
    for r in range(1, 5):
        for rot in rotations[r % 2]:
            x0 = x0 + x1
            x1 = ((x1 << _U32(rot)) | (x1 >> _U32(32 - rot))) ^ x0
        x0 = x0 + _KS[(r + 1) % 3]
        x1 = x1 + _U32(_KS[(r + 2) % 3] + _U32(r + 1))
    bits = x0 ^ x1

    fb = lax.bitcast_convert_type(
        (bits >> _U32(9)) | _U32(0x3F800000), jnp.float32) - jnp.float32(1.0)
    u = jnp.maximum(fb, _TINY)
    g = -jnp.log(-jnp.log(u))

    v = g + logits
    mx = jnp.max(v, axis=0, keepdims=True)
    picked = jnp.where(v == mx, noisy_ref[...], jnp.float32(np.inf))
    out_ref[0] = jnp.min(picked, axis=0, keepdims=True)


def kernel(x, noise_matrix):
    nblk = _NPIX // _NB
    lognm = jnp.log(noise_matrix)
    lower = jnp.asarray(np.broadcast_to(_LOWER[:, None], (_C, _NB)))
    upper = jnp.asarray(np.broadcast_to(_UPPER[:, None], (_C, _NB)))
    noisy = jnp.asarray(np.broadcast_to(_NOISY[:, None], (_C, _NB)))
    ctab = jnp.asarray(
        101 * np.arange(_NB, dtype=np.uint32)[None, :]
        + np.arange(_C, dtype=np.uint32)[:, None]
        + np.uint32(42)
    )

    ndev = jax.device_count()
    if ndev > 1 and x.shape[0] % ndev == 0 and jax.process_count() == 1:
        mesh = jax.sharding.Mesh(np.array(jax.devices()), ("d",))
        nloc = nblk // ndev
        imgs = x.shape[0] // ndev

        def _shard_fn(x_s, lognm_s, lower_s, upper_s, noisy_s, ctab_s):
            idx = lax.axis_index("d")
            base = idx.astype(jnp.uint32) * np.uint32(nloc * _NB * _C)
            x_loc = lax.dynamic_slice_in_dim(x_s, idx * imgs, imgs, axis=0)
            out_s = _call_pallas(
                x_loc.reshape(nloc, 1, _NB), lognm_s, lower_s, upper_s,
                noisy_s, ctab_s + base, nloc,
            )
            return out_s.reshape(imgs, *x.shape[1:])

        P = jax.sharding.PartitionSpec
        return jax.shard_map(
            _shard_fn,
            mesh=mesh,
            in_specs=(
                P(None, None, None), P(None, None), P(None, None),
                P(None, None), P(None, None), P(None, None),
            ),
            out_specs=P("d", None, None),
            check_vma=False,
        )(x, lognm, lower, upper, noisy, ctab)
    xb = x.reshape(nblk, 1, _NB)
    out = _call_pallas(xb, lognm, lower, upper, noisy, ctab, nblk)
    return out.reshape(x.shape)


def _call_pallas(xb, lognm, lower, upper, noisy, ctab, nblk):
    return pl.pallas_call(
        _kern,
        grid=(nblk,),
        in_specs=[
            pl.BlockSpec((1, 1, _NB), lambda i: (i, 0, 0)),
            pl.BlockSpec((_C, _C), lambda i: (0, 0)),
            pl.BlockSpec((_C, _NB), lambda i: (0, 0)),
            pl.BlockSpec((_C, _NB), lambda i: (0, 0)),
            pl.BlockSpec((_C, _NB), lambda i: (0, 0)),
            pl.BlockSpec((_C, _NB), lambda i: (0, 0)),
        ],
        out_specs=pl.BlockSpec((1, 1, _NB), lambda i: (i, 0, 0)),
        out_shape=jax.ShapeDtypeStruct((nblk, 1, _NB), jnp.float32),
        compiler_params=pltpu.CompilerParams(
            dimension_semantics=("arbitrary",),
        ),
    )(xb, lognm, lower, upper, noisy, ctab)
